# baseline (speedup 1.0000x reference)
import functools

import jax
import jax.numpy as jnp
from jax import lax
from jax.experimental import pallas as pl
from jax.experimental.pallas import tpu as pltpu
from jax.experimental.pallas import tpu_sc as plsc

B = 4096
L = 200
EMB = 64
VLM = 768
TXT = 512

NC = 2
NS = 16
NW = NC * NS
BLK = B // NW
NBUF = 2


def _projection_t(vlm_emb, text_emb, W1a, W1b, b1_2d):
    blk = 512
    grid = (B // blk,)

    def body(vlm_ref, txt_ref, wa_ref, wb_ref, b1_ref, o_ref):
        acc = jnp.dot(vlm_ref[...], wa_ref[...], preferred_element_type=jnp.float32)
        acc = acc + jnp.dot(txt_ref[...], wb_ref[...], preferred_element_type=jnp.float32)
        o_ref[...] = (acc + b1_ref[...]).T

    return pl.pallas_call(
        body,
        grid=grid,
        in_specs=[
            pl.BlockSpec((blk, VLM), lambda i: (i, 0)),
            pl.BlockSpec((blk, TXT), lambda i: (i, 0)),
            pl.BlockSpec((VLM, EMB), lambda i: (0, 0)),
            pl.BlockSpec((TXT, EMB), lambda i: (0, 0)),
            pl.BlockSpec((1, EMB), lambda i: (0, 0)),
        ],
        out_specs=pl.BlockSpec((EMB, blk), lambda i: (0, i)),
        out_shape=jax.ShapeDtypeStruct((EMB, B), jnp.float32),
    )(vlm_emb, text_emb, W1a, W1b, b1_2d)


def _gather_add(idx2, off, proj_t, t128):
    mesh = plsc.VectorSubcoreMesh(core_axis_name="c", subcore_axis_name="s")

    @functools.partial(
        pl.kernel,
        out_type=jax.ShapeDtypeStruct((L, EMB, B), jnp.float32),
        mesh=mesh,
        scratch_types=[
            pltpu.VMEM((L, BLK), jnp.int32),
            pltpu.VMEM((L, BLK), jnp.int32),
            pltpu.VMEM((EMB, BLK), jnp.float32),
            pltpu.VMEM((NBUF, BLK, 128), jnp.float32),
            pltpu.VMEM((NBUF, EMB, BLK), jnp.float32),
            pltpu.SemaphoreType.DMA((NBUF,)),
            pltpu.SemaphoreType.DMA((NBUF,)),
        ],
        compiler_params=pltpu.CompilerParams(needs_layout_passes=False),
    )
    def k(idx_hbm, off_hbm, proj_hbm, table_hbm, out_hbm,
          idx_v, off_v, ptv, g_v, o_v, gsem, osem):
        wid = lax.axis_index("s") * NC + lax.axis_index("c")
        base = wid * BLK

        pltpu.sync_copy(idx_hbm.at[:, pl.ds(base, BLK)], idx_v)
        pltpu.sync_copy(off_hbm.at[:, pl.ds(base, BLK)], off_v)
        pltpu.sync_copy(proj_hbm.at[:, pl.ds(base, BLK)], ptv)

        def start_gather(l, buf):
            pltpu.async_copy(table_hbm.at[idx_v.at[l]], g_v.at[buf], gsem.at[buf])

        def wait_gather(l, buf):
            pltpu.make_async_copy(
                table_hbm.at[idx_v.at[l]], g_v.at[buf], gsem.at[buf]
            ).wait()

        def wait_write(l, buf):
            pltpu.make_async_copy(
                o_v.at[buf], out_hbm.at[l, :, pl.ds(base, BLK)], osem.at[buf]
            ).wait()

        start_gather(0, 0)
        start_gather(1, 1)

        lane = lax.iota(jnp.int32, 16)
        rowg = [lane + 16 * g for g in range(8)]

        @pl.loop(0, L // NBUF)
        def _t(t):
            for kk in range(NBUF):
                buf = kk
                l = t * NBUF + kk

                wait_gather(l, buf)

                @pl.when(l >= 2)
                def _drain():
                    wait_write(l - 2, buf)

                ovg = [off_v[l, pl.ds(16 * g, 16)] for g in range(8)]

                @pl.loop(0, EMB)
                def _e(e):
                    for g in range(8):
                        v = plsc.load_gather(g_v.at[buf], [rowg[g], ovg[g] + e])
                        o_v[buf, e, pl.ds(16 * g, 16)] = v + ptv[e, pl.ds(16 * g, 16)]

                @pl.when(l + 2 < L)
                def _pf():
                    start_gather(l + 2, buf)

                pltpu.async_copy(
                    o_v.at[buf], out_hbm.at[l, :, pl.ds(base, BLK)], osem.at[buf]
                )

        wait_write(L - 2, 0)
        wait_write(L - 1, 1)

    return k(idx2, off, proj_t, t128)


def kernel(vlm_emb, text_emb, input_ids, embedding, W1, b1):
    W1a = W1[:VLM]
    W1b = W1[VLM:]
    proj_t = _projection_t(vlm_emb, text_emb, W1a, W1b, b1.reshape(1, EMB))

    ids_l = input_ids.astype(jnp.int32).T
    idx2 = ids_l >> 1
    off = (ids_l & 1) << 6
    t128 = embedding.reshape(500000, 128)

    out_leb = _gather_add(idx2, off, proj_t, t128)
    return out_leb.transpose(2, 0, 1)


# speedup vs baseline: 1.4928x; 1.4928x over previous
import functools

import jax
import jax.numpy as jnp
from jax import lax
from jax.experimental import pallas as pl
from jax.experimental.pallas import tpu as pltpu
from jax.experimental.pallas import tpu_sc as plsc

B = 4096
L = 200
EMB = 64
VLM = 768
TXT = 512

NC = 2
NS = 16
NW = NC * NS
BLK = B // NW
NBUF = 2


def _projection_t(vlm_emb, text_emb, W1a, W1b, b1_2d):
    blk = 512
    grid = (B // blk,)

    def body(vlm_ref, txt_ref, wa_ref, wb_ref, b1_ref, o_ref):
        acc = jnp.dot(vlm_ref[...], wa_ref[...], preferred_element_type=jnp.float32)
        acc = acc + jnp.dot(txt_ref[...], wb_ref[...], preferred_element_type=jnp.float32)
        o_ref[...] = (acc + b1_ref[...]).T

    return pl.pallas_call(
        body,
        grid=grid,
        in_specs=[
            pl.BlockSpec((blk, VLM), lambda i: (i, 0)),
            pl.BlockSpec((blk, TXT), lambda i: (i, 0)),
            pl.BlockSpec((VLM, EMB), lambda i: (0, 0)),
            pl.BlockSpec((TXT, EMB), lambda i: (0, 0)),
            pl.BlockSpec((1, EMB), lambda i: (0, 0)),
        ],
        out_specs=pl.BlockSpec((EMB, blk), lambda i: (0, i)),
        out_shape=jax.ShapeDtypeStruct((EMB, B), jnp.float32),
    )(vlm_emb, text_emb, W1a, W1b, b1_2d)


def _gather_add(idx2, off, proj_t, t128):
    mesh = plsc.VectorSubcoreMesh(core_axis_name="c", subcore_axis_name="s")

    @functools.partial(
        pl.kernel,
        out_type=jax.ShapeDtypeStruct((L, EMB, B), jnp.float32),
        mesh=mesh,
        scratch_types=[
            pltpu.VMEM((L, BLK), jnp.int32),
            pltpu.VMEM((L, BLK), jnp.int32),
            pltpu.VMEM((EMB, BLK), jnp.float32),
            pltpu.VMEM((NBUF, BLK, 128), jnp.float32),
            pltpu.VMEM((NBUF, EMB, BLK), jnp.float32),
            pltpu.SemaphoreType.DMA((NBUF,)),
            pltpu.SemaphoreType.DMA((NBUF,)),
        ],
        compiler_params=pltpu.CompilerParams(needs_layout_passes=False),
    )
    def k(idx_hbm, off_hbm, proj_hbm, table_hbm, out_hbm,
          idx_v, off_v, ptv, g_v, o_v, gsem, osem):
        wid = lax.axis_index("s") * NC + lax.axis_index("c")
        base = wid * BLK

        pltpu.sync_copy(idx_hbm.at[:, pl.ds(base, BLK)], idx_v)
        pltpu.sync_copy(off_hbm.at[:, pl.ds(base, BLK)], off_v)
        pltpu.sync_copy(proj_hbm.at[:, pl.ds(base, BLK)], ptv)

        def start_gather(l, buf):
            pltpu.async_copy(table_hbm.at[idx_v.at[l]], g_v.at[buf], gsem.at[buf])

        def wait_gather(l, buf):
            pltpu.make_async_copy(
                table_hbm.at[idx_v.at[l]], g_v.at[buf], gsem.at[buf]
            ).wait()

        def wait_write(l, buf):
            pltpu.make_async_copy(
                o_v.at[buf], out_hbm.at[l, :, pl.ds(base, BLK)], osem.at[buf]
            ).wait()

        start_gather(0, 0)
        start_gather(1, 1)

        lane = lax.iota(jnp.int32, 16)
        rowg = [lane + 16 * g for g in range(8)]

        @pl.loop(0, L // NBUF)
        def _t(t):
            for kk in range(NBUF):
                buf = kk
                l = t * NBUF + kk

                wait_gather(l, buf)

                @pl.when(l >= 2)
                def _drain():
                    wait_write(l - 2, buf)

                ovg = [off_v[l, pl.ds(16 * g, 16)] for g in range(8)]

                @plsc.parallel_loop(0, EMB, unroll=4)
                def _e(e):
                    for g in range(8):
                        v = plsc.load_gather(g_v.at[buf], [rowg[g], ovg[g] + e])
                        o_v[buf, e, pl.ds(16 * g, 16)] = v + ptv[e, pl.ds(16 * g, 16)]

                @pl.when(l + 2 < L)
                def _pf():
                    start_gather(l + 2, buf)

                pltpu.async_copy(
                    o_v.at[buf], out_hbm.at[l, :, pl.ds(base, BLK)], osem.at[buf]
                )

        wait_write(L - 2, 0)
        wait_write(L - 1, 1)

    return k(idx2, off, proj_t, t128)


def kernel(vlm_emb, text_emb, input_ids, embedding, W1, b1):
    W1a = W1[:VLM]
    W1b = W1[VLM:]
    proj_t = _projection_t(vlm_emb, text_emb, W1a, W1b, b1.reshape(1, EMB))

    ids_l = input_ids.astype(jnp.int32).T
    idx2 = ids_l >> 1
    off = (ids_l & 1) << 6
    t128 = embedding.reshape(500000, 128)

    out_leb = _gather_add(idx2, off, proj_t, t128)
    return out_leb.transpose(2, 0, 1)


# speedup vs baseline: 1.5203x; 1.0184x over previous
import functools

import jax
import jax.numpy as jnp
from jax import lax
from jax.experimental import pallas as pl
from jax.experimental.pallas import tpu as pltpu
from jax.experimental.pallas import tpu_sc as plsc

B = 4096
L = 200
LP = 256
EMB = 64
VLM = 768
TXT = 512

NC = 2
NS = 16
NW = NC * NS
BPW = B // NW
NP = BPW // 2
C0 = 128
C1 = L - C0


def _projection(vlm_emb, text_emb, W1a, W1b, b1_2d):
    blk = 512
    grid = (B // blk,)

    def body(vlm_ref, txt_ref, wa_ref, wb_ref, b1_ref, o_ref):
        acc = jnp.dot(vlm_ref[...], wa_ref[...], preferred_element_type=jnp.float32)
        acc = acc + jnp.dot(txt_ref[...], wb_ref[...], preferred_element_type=jnp.float32)
        o_ref[...] = acc + b1_ref[...]

    return pl.pallas_call(
        body,
        grid=grid,
        in_specs=[
            pl.BlockSpec((blk, VLM), lambda i: (i, 0)),
            pl.BlockSpec((blk, TXT), lambda i: (i, 0)),
            pl.BlockSpec((VLM, EMB), lambda i: (0, 0)),
            pl.BlockSpec((TXT, EMB), lambda i: (0, 0)),
            pl.BlockSpec((1, EMB), lambda i: (0, 0)),
        ],
        out_specs=pl.BlockSpec((blk, EMB), lambda i: (i, 0)),
        out_shape=jax.ShapeDtypeStruct((B, EMB), jnp.float32),
    )(vlm_emb, text_emb, W1a, W1b, b1_2d)


def _gather_add(raw_flat, proj_flat, t128):
    mesh = plsc.VectorSubcoreMesh(core_axis_name="c", subcore_axis_name="s")

    @functools.partial(
        pl.kernel,
        out_type=jax.ShapeDtypeStruct((B // 2, 25, 8, 128), jnp.float32),
        mesh=mesh,
        scratch_types=[
            pltpu.VMEM((2 * LP,), jnp.int32),
            pltpu.VMEM((2 * LP,), jnp.int32),
            pltpu.VMEM((BPW * EMB,), jnp.float32),
            pltpu.VMEM((2, LP), jnp.int32),
            pltpu.VMEM((2, L, 128), jnp.float32),
            pltpu.VMEM((2, 25, 8, 128), jnp.float32),
            pltpu.SemaphoreType.DMA((2,)),
            pltpu.SemaphoreType.DMA((2,)),
            pltpu.SemaphoreType.DMA((2,)),
        ],
    )
    def k(raw_hbm, proj_hbm, table_hbm, out_hbm,
          raw_v0, raw_v1, pv, idx_v, g_v, w_v, rsem, gsem, osem):
        raw_bufs = (raw_v0, raw_v1)
        wid = lax.axis_index("s") * NC + lax.axis_index("c")
        base = wid * BPW
        pbase = wid * NP

        pltpu.sync_copy(proj_hbm.at[pl.ds(base * EMB, BPW * EMB)], pv)

        def start_raw(p, buf):
            pltpu.async_copy(
                raw_hbm.at[pl.ds((base + 2 * p) * LP, 2 * LP)],
                raw_bufs[buf],
                rsem.at[buf],
            )

        def wait_raw(p, buf):
            pltpu.make_async_copy(
                raw_hbm.at[pl.ds((base + 2 * p) * LP, 2 * LP)],
                raw_bufs[buf],
                rsem.at[buf],
            ).wait()

        def fill_idx(rbuf, kk, buf):
            @pl.loop(0, 13)
            def _m(m):
                idx_v[buf, pl.ds(16 * m, 16)] = (
                    raw_bufs[rbuf][pl.ds(kk * LP + 16 * m, 16)] >> 1
                )

        def start_gather(buf):
            pltpu.async_copy(
                table_hbm.at[idx_v.at[buf, pl.ds(0, C0)]],
                g_v.at[buf, pl.ds(0, C0)],
                gsem.at[buf],
            )
            pltpu.async_copy(
                table_hbm.at[idx_v.at[buf, pl.ds(C0, C1)]],
                g_v.at[buf, pl.ds(C0, C1)],
                gsem.at[buf],
            )

        def wait_gather(buf):
            pltpu.make_async_copy(
                table_hbm.at[idx_v.at[buf, pl.ds(0, C0)]],
                g_v.at[buf, pl.ds(0, C0)],
                gsem.at[buf],
            ).wait()
            pltpu.make_async_copy(
                table_hbm.at[idx_v.at[buf, pl.ds(C0, C1)]],
                g_v.at[buf, pl.ds(C0, C1)],
                gsem.at[buf],
            ).wait()

        def wait_write(p, buf):
            pltpu.make_async_copy(
                w_v.at[buf], out_hbm.at[pbase + p], osem.at[buf]
            ).wait()

        pltpu.sync_copy(raw_hbm.at[pl.ds(base * LP, 2 * LP)], raw_v0)
        fill_idx(0, 0, 0)
        start_gather(0)
        fill_idx(0, 1, 1)
        start_gather(1)
        start_raw(1, 1)

        @pl.loop(0, NP // 2)
        def _t(t):
            for pp in range(2):
                wbuf = pp
                p = t * 2 + pp
                rbuf = pp

                @pl.when(p >= 2)
                def _drain():
                    wait_write(p - 2, wbuf)

                for kk in range(2):
                    gbuf = kk
                    i = 2 * p + kk

                    wait_gather(gbuf)

                    pj = [pv[pl.ds(i * EMB + 16 * j, 16)] for j in range(4)]

                    def compact_rows(m, nrows):
                        ob = raw_bufs[rbuf][pl.ds(kk * LP + 16 * m, 16)] & 1
                        for rr in range(nrows):
                            r = 16 * m + rr
                            hi = ob[rr] != 0
                            c = 100 * kk + rr // 2
                            toff, row = c // 8, c % 8
                            for j in range(4):
                                vlo = g_v[gbuf, r, pl.ds(16 * j, 16)]
                                vhi = g_v[gbuf, r, pl.ds(64 + 16 * j, 16)]
                                w_v[wbuf, m + toff, row,
                                    pl.ds(64 * (rr & 1) + 16 * j, 16)] = (
                                    jnp.where(hi, vhi, vlo) + pj[j]
                                )

                    @pl.loop(0, 12)
                    def _r(m):
                        compact_rows(m, 16)

                    compact_rows(12, 8)

                    @pl.when(i + 2 < BPW)
                    def _pf():
                        if kk == 0:
                            wait_raw(p + 1, 1 - rbuf)
                        fill_idx(1 - rbuf, kk, gbuf)
                        start_gather(gbuf)

                @pl.when(p + 2 < NP)
                def _praw():
                    start_raw(p + 2, rbuf)

                pltpu.async_copy(
                    w_v.at[wbuf], out_hbm.at[pbase + p], osem.at[wbuf]
                )

        wait_write(NP - 2, 0)
        wait_write(NP - 1, 1)

    return k(raw_flat, proj_flat, t128)


def kernel(vlm_emb, text_emb, input_ids, embedding, W1, b1):
    W1a = W1[:VLM]
    W1b = W1[VLM:]
    proj = _projection(vlm_emb, text_emb, W1a, W1b, b1.reshape(1, EMB))

    ids32 = input_ids.astype(jnp.int32)
    raw_flat = jnp.pad(ids32, ((0, 0), (0, LP - L))).reshape(-1)
    t128 = embedding.reshape(500000, 128)

    out4 = _gather_add(raw_flat, proj.reshape(-1), t128)
    return out4.reshape(B, L, EMB)


# speedup vs baseline: 2.6802x; 1.7629x over previous
import functools

import jax
import jax.numpy as jnp
from jax import lax
from jax.experimental import pallas as pl
from jax.experimental.pallas import tpu as pltpu
from jax.experimental.pallas import tpu_sc as plsc

B = 4096
L = 200
EMB = 64
VLM = 768
TXT = 512
VOCAB = 1000000

NC = 2
NS = 16
NW = NC * NS
BPW = B // NW
NBUF = 4
C0 = 128
C1 = L - C0

TBLK = 8192


NBLK = (VOCAB + 2 * TBLK - 1) // (2 * TBLK)
TROWS = 2 * NBLK * TBLK


def _detile_table(embT):
    def body(ina_ref, inb_ref, o_ref):
        o_ref[...] = jnp.concatenate(
            [ina_ref[...].T, inb_ref[...].T], axis=1
        )

    return pl.pallas_call(
        body,
        grid=(NBLK,),
        in_specs=[
            pl.BlockSpec((EMB, TBLK), lambda i: (0, 2 * i)),
            pl.BlockSpec((EMB, TBLK), lambda i: (0, jnp.minimum(2 * i + 1, 122))),
        ],
        out_specs=pl.BlockSpec((TBLK, 2 * EMB), lambda i: (i, 0)),
        out_shape=jax.ShapeDtypeStruct((NBLK * TBLK, 2 * EMB), jnp.float32),
    )(embT, embT)


def _transpose_out(o2):
    def body(in_ref, o_ref):
        o_ref[...] = in_ref[...].T

    RB, CB = 512, 2560
    return pl.pallas_call(
        body,
        grid=(B // RB, (L * EMB) // CB),
        in_specs=[pl.BlockSpec((RB, CB), lambda i, j: (i, j))],
        out_specs=pl.BlockSpec((CB, RB), lambda i, j: (j, i)),
        out_shape=jax.ShapeDtypeStruct((L * EMB, B), jnp.float32),
    )(o2)


def _projection(vlm_emb, text_emb, W1a, W1b, b1_2d):
    blk = 512
    grid = (B // blk,)

    def body(vlm_ref, txt_ref, wa_ref, wb_ref, b1_ref, o_ref):
        acc = jnp.dot(vlm_ref[...], wa_ref[...], preferred_element_type=jnp.float32)
        acc = acc + jnp.dot(txt_ref[...], wb_ref[...], preferred_element_type=jnp.float32)
        o_ref[...] = acc + b1_ref[...]

    return pl.pallas_call(
        body,
        grid=grid,
        in_specs=[
            pl.BlockSpec((blk, VLM), lambda i: (i, 0)),
            pl.BlockSpec((blk, TXT), lambda i: (i, 0)),
            pl.BlockSpec((VLM, EMB), lambda i: (0, 0)),
            pl.BlockSpec((TXT, EMB), lambda i: (0, 0)),
            pl.BlockSpec((1, EMB), lambda i: (0, 0)),
        ],
        out_specs=pl.BlockSpec((blk, EMB), lambda i: (i, 0)),
        out_shape=jax.ShapeDtypeStruct((B, EMB), jnp.float32),
    )(vlm_emb, text_emb, W1a, W1b, b1_2d)


def _gather_add(ids, proj, table):
    mesh = plsc.VectorSubcoreMesh(core_axis_name="c", subcore_axis_name="s")

    @functools.partial(
        pl.kernel,
        out_type=jax.ShapeDtypeStruct((B, L, EMB), jnp.float32),
        mesh=mesh,
        scratch_types=[
            pltpu.VMEM((BPW, L), jnp.int32),
            pltpu.VMEM((BPW, EMB), jnp.float32),
            pltpu.VMEM((NBUF, L, EMB), jnp.float32),
            pltpu.SemaphoreType.DMA((NBUF,)),
            pltpu.SemaphoreType.DMA((NBUF,)),
        ],
        compiler_params=pltpu.CompilerParams(use_tc_tiling_on_sc=False),
    )
    def k(ids_hbm, proj_hbm, table_hbm, out_hbm, idx_v, projs_v, rows_v, gsem, osem):
        wid = lax.axis_index("s") * NC + lax.axis_index("c")
        base = wid * BPW

        pltpu.sync_copy(ids_hbm.at[pl.ds(base, BPW)], idx_v)
        pltpu.sync_copy(proj_hbm.at[pl.ds(base, BPW)], projs_v)

        def start_gather(i, buf):
            pltpu.async_copy(
                table_hbm.at[idx_v.at[i, pl.ds(0, C0)]],
                rows_v.at[buf, pl.ds(0, C0)],
                gsem.at[buf],
            )
            pltpu.async_copy(
                table_hbm.at[idx_v.at[i, pl.ds(C0, C1)]],
                rows_v.at[buf, pl.ds(C0, C1)],
                gsem.at[buf],
            )

        def wait_gather(i, buf):
            pltpu.make_async_copy(
                table_hbm.at[idx_v.at[i, pl.ds(0, C0)]],
                rows_v.at[buf, pl.ds(0, C0)],
                gsem.at[buf],
            ).wait()
            pltpu.make_async_copy(
                table_hbm.at[idx_v.at[i, pl.ds(C0, C1)]],
                rows_v.at[buf, pl.ds(C0, C1)],
                gsem.at[buf],
            ).wait()

        def wait_write(i, buf):
            pltpu.make_async_copy(
                rows_v.at[buf], out_hbm.at[base + i], osem.at[buf]
            ).wait()

        start_gather(0, 0)
        start_gather(1, 1)

        @pl.loop(0, BPW // NBUF)
        def _t(t):
            for kk in range(NBUF):
                i = t * NBUF + kk
                buf = kk
                nbuf = (kk + 2) % NBUF

                @pl.when(i + 2 < BPW)
                def _pf():
                    @pl.when(i >= 2)
                    def _drain():
                        wait_write(i - 2, nbuf)

                    start_gather(i + 2, nbuf)

                wait_gather(i, buf)

                pj0 = projs_v[i, pl.ds(0, 16)]
                pj1 = projs_v[i, pl.ds(16, 16)]
                pj2 = projs_v[i, pl.ds(32, 16)]
                pj3 = projs_v[i, pl.ds(48, 16)]

                @pl.loop(0, L // 4)
                def _r(r4):
                    for rr in range(4):
                        r = r4 * 4 + rr
                        rows_v[buf, r, pl.ds(0, 16)] += pj0
                        rows_v[buf, r, pl.ds(16, 16)] += pj1
                        rows_v[buf, r, pl.ds(32, 16)] += pj2
                        rows_v[buf, r, pl.ds(48, 16)] += pj3

                pltpu.async_copy(rows_v.at[buf], out_hbm.at[base + i], osem.at[buf])

        for kk in range(NBUF):
            wait_write(BPW - NBUF + kk, kk)

    return k(ids, proj, table)


def kernel(vlm_emb, text_emb, input_ids, embedding, W1, b1):
    W1a = W1[:VLM]
    W1b = W1[VLM:]
    proj = _projection(vlm_emb, text_emb, W1a, W1b, b1.reshape(1, EMB))

    table_lin = _detile_table(embedding.T).reshape(TROWS, EMB)
    ids32 = input_ids.astype(jnp.int32)
    g = ids32 >> 13
    idx2 = ((g >> 1) << 14) | ((ids32 & 8191) << 1) | (g & 1)
    out = _gather_add(idx2, proj, table_lin)

    flat = jax.lax.optimization_barrier(out.reshape(B * L * EMB))
    o2 = flat.reshape(B, L * EMB)
    o3 = _transpose_out(o2)
    return o3.reshape(L, EMB, B).transpose(2, 0, 1)
